# baseline (speedup 1.0000x reference)
import functools

import jax
import jax.numpy as jnp
from jax import lax
from jax.experimental import pallas as pl
from jax.experimental.pallas import tpu as pltpu
from jax.experimental.pallas import tpu_sc as plsc

H, W = 256, 832
NQ = (H // 2) * (W // 4)
NK = NQ // 4
KNN = 8
NPAIR = NQ * KNN

QB = 256
PB = 128


def _dot3(x, w):
    return (x[:, 0:1] * w[0:1, :] + x[:, 1:2] * w[1:2, :]
            + x[:, 2:3] * w[2:3, :])


def _p1_body(col_ref, posk_ref, wf_ref, bf_ref, a_ref, b1m_ref,
             b1_ref, g1t_ref):
    f1 = jnp.maximum(_dot3(col_ref[...], wf_ref[...]) + bf_ref[...], 0.0)
    g1t_ref[...] = (_dot3(posk_ref[...], a_ref[...])
                    + jnp.dot(f1, b1m_ref[...],
                              preferred_element_type=jnp.float32,
                              precision=jax.lax.Precision.HIGHEST)
                    + b1_ref[...])


def _p1(col_l1, pos_l1, w_feat, b_feat, w1a, w1b, b1):
    return pl.pallas_call(
        _p1_body,
        out_shape=jax.ShapeDtypeStruct((NK, 64), jnp.float32),
    )(col_l1, pos_l1, w_feat, b_feat.reshape(1, 64), w1a, w1b,
      b1.reshape(1, 64))


def _p2_body(q_ref, kt_ref, a_ref, idx_ref, d2_ref, qa_ref):
    q = q_ref[...]
    kt = kt_ref[...]
    qa_ref[...] = _dot3(q, a_ref[...])
    q2 = jnp.sum(q * q, axis=1, keepdims=True)
    k2 = jnp.sum(kt * kt, axis=0, keepdims=True)
    d2 = q2 + k2 - 2.0 * jnp.dot(q, kt, preferred_element_type=jnp.float32)
    cols = lax.broadcasted_iota(jnp.int32, (QB, NK), 1)
    big_i = jnp.int32(2 ** 30)
    ixs = []
    d2s = []
    for _ in range(KNN):
        m = jnp.min(d2, axis=1, keepdims=True)
        ix = jnp.min(jnp.where(d2 == m, cols, big_i), axis=1, keepdims=True)
        ixs.append(ix)
        d2s.append(m)
        d2 = jnp.where(cols == ix, jnp.inf, d2)
    idx_ref[...] = jnp.concatenate(ixs, axis=1)
    d2_ref[...] = jnp.concatenate(d2s, axis=1)


def _p2(pos_l0, kt, w1a):
    grid = NQ // QB
    return pl.pallas_call(
        _p2_body,
        grid=(grid,),
        in_specs=[
            pl.BlockSpec((QB, 3), lambda i: (i, 0)),
            pl.BlockSpec((3, NK), lambda i: (0, 0)),
            pl.BlockSpec((3, 64), lambda i: (0, 0)),
        ],
        out_specs=(
            pl.BlockSpec((QB, KNN), lambda i: (i, 0)),
            pl.BlockSpec((QB, KNN), lambda i: (i, 0)),
            pl.BlockSpec((QB, 64), lambda i: (i, 0)),
        ),
        out_shape=(
            jax.ShapeDtypeStruct((NQ, KNN), jnp.int32),
            jax.ShapeDtypeStruct((NQ, KNN), jnp.float32),
            jax.ShapeDtypeStruct((NQ, 64), jnp.float32),
        ),
    )(pos_l0, kt, w1a)


def _p3(idx_flat, table):
    info = plsc.get_sparse_core_info()
    nw = info.num_cores * info.num_subcores
    ch = 128
    per_w = NPAIR // (nw * ch)
    mesh = plsc.VectorSubcoreMesh(core_axis_name="c", subcore_axis_name="s")

    @functools.partial(
        pl.kernel,
        out_type=jax.ShapeDtypeStruct((NPAIR, 64), jnp.float32),
        mesh=mesh,
        scratch_types=[
            pltpu.VMEM((ch,), jnp.int32),
            pltpu.VMEM((ch, 64), jnp.float32),
            pltpu.SemaphoreType.DMA,
        ],
        compiler_params=pltpu.CompilerParams(use_tc_tiling_on_sc=False),
    )
    def gather_rows(idx_hbm, tab_hbm, out_hbm, idx_v, rows_v, sem):
        wid = lax.axis_index("s") * info.num_cores + lax.axis_index("c")

        def body(j, carry):
            base = (wid * per_w + j) * ch
            pltpu.sync_copy(idx_hbm.at[pl.ds(base, ch)], idx_v)
            pltpu.async_copy(tab_hbm.at[idx_v], rows_v, sem).wait()
            pltpu.sync_copy(rows_v, out_hbm.at[pl.ds(base, ch)])
            return carry

        lax.fori_loop(0, per_w, body, 0)

    return gather_rows(idx_flat, table)


def _p4_body(gg_ref, qa_ref, d2_ref, w2_ref, b2_ref, w3_ref, b3_ref,
             w4_ref, b4_ref, out_ref):
    qa = qa_ref[...]
    pooled = None
    for n in range(KNN):
        gn = gg_ref[n]
        h1 = jnp.maximum(gn - qa, 0.0)
        h2 = jnp.maximum(jnp.dot(h1, w2_ref[...],
                                 preferred_element_type=jnp.float32, precision=jax.lax.Precision.HIGHEST)
                         + b2_ref[...], 0.0)
        h3 = jnp.maximum(jnp.dot(h2, w3_ref[...],
                                 preferred_element_type=jnp.float32, precision=jax.lax.Precision.HIGHEST)
                         + b3_ref[...], 0.0)
        hm = jnp.where(d2_ref[n] <= 100.0 * 100.0, h3, -1e9)
        pooled = hm if pooled is None else jnp.maximum(pooled, hm)
    out_ref[...] = jnp.maximum(jnp.dot(pooled, w4_ref[...],
                                       preferred_element_type=jnp.float32, precision=jax.lax.Precision.HIGHEST)
                               + b4_ref[...], 0.0)


def _p4(gathered, qa, d2t, w2, b2, w3, b3, w4, b4):
    grid = NQ // PB
    return pl.pallas_call(
        _p4_body,
        grid=(grid,),
        in_specs=[
            pl.BlockSpec((KNN, PB, 64), lambda i: (0, i, 0)),
            pl.BlockSpec((PB, 64), lambda i: (i, 0)),
            pl.BlockSpec((KNN, PB, 1), lambda i: (0, i, 0)),
            pl.BlockSpec((64, 64), lambda i: (0, 0)),
            pl.BlockSpec((1, 64), lambda i: (0, 0)),
            pl.BlockSpec((64, 32), lambda i: (0, 0)),
            pl.BlockSpec((1, 32), lambda i: (0, 0)),
            pl.BlockSpec((32, 32), lambda i: (0, 0)),
            pl.BlockSpec((1, 32), lambda i: (0, 0)),
        ],
        out_specs=pl.BlockSpec((PB, 32), lambda i: (i, 0)),
        out_shape=jax.ShapeDtypeStruct((NQ, 32), jnp.float32),
    )(gathered, qa, d2t, w2, b2.reshape(1, 64), w3, b3.reshape(1, 32),
      w4, b4.reshape(1, 32))


def kernel(pos1_bhw3, pos2_bhw3, color1, color2, intrinsics, occ_mask,
           W_feat, b_feat, W1, b1, W2, b2, W3, b3, W4, b4):
    pos_l0_img = pos1_bhw3[0, ::2, ::4]
    pos_l0 = pos_l0_img.reshape(-1, 3)
    pos_l1 = pos_l0_img[::2, ::2].reshape(-1, 3)
    col_l1 = color1[0, ::2, ::4][::2, ::2].reshape(-1, 3)

    w1a = W1[:3]
    w1b = W1[3:]

    g1t = _p1(col_l1, pos_l1, W_feat, b_feat, w1a, w1b, b1)
    idx, d2t, qa = _p2(pos_l0, pos_l1.T, w1a)
    idx_nm = idx.T.reshape(-1)
    d2_nm = d2t.T.reshape(KNN, NQ, 1)
    gathered = _p3(idx_nm, g1t).reshape(KNN, NQ, 64)
    out = _p4(gathered, qa, d2_nm, W2, b2, W3, b3, W4, b4)
    return out.reshape(1, NQ, 32)


# speedup vs baseline: 1.1334x; 1.1334x over previous
import functools

import jax
import jax.numpy as jnp
from jax import lax
from jax.experimental import pallas as pl
from jax.experimental.pallas import tpu as pltpu
from jax.experimental.pallas import tpu_sc as plsc

H, W = 256, 832
NQ = (H // 2) * (W // 4)
NK = NQ // 4
KNN = 8
NPAIR = NQ * KNN

QB = 256
PB = 128


def _dot3(x, w):
    return (x[:, 0:1] * w[0:1, :] + x[:, 1:2] * w[1:2, :]
            + x[:, 2:3] * w[2:3, :])


def _p1_body(col_ref, posk_ref, wf_ref, bf_ref, a_ref, b1m_ref,
             b1_ref, g1t_ref):
    f1 = jnp.maximum(_dot3(col_ref[...], wf_ref[...]) + bf_ref[...], 0.0)
    g1t_ref[...] = (_dot3(posk_ref[...], a_ref[...])
                    + jnp.dot(f1, b1m_ref[...],
                              preferred_element_type=jnp.float32,
                              precision=jax.lax.Precision.HIGHEST)
                    + b1_ref[...])


def _p1(col_l1, pos_l1, w_feat, b_feat, w1a, w1b, b1):
    return pl.pallas_call(
        _p1_body,
        out_shape=jax.ShapeDtypeStruct((NK, 64), jnp.float32),
    )(col_l1, pos_l1, w_feat, b_feat.reshape(1, 64), w1a, w1b,
      b1.reshape(1, 64))


_LANES = 128
_DEPTH = NK // _LANES


def _p2_body(q_ref, kt_ref, a_ref, idx_ref, d2_ref, qa_ref):
    q = q_ref[...]
    kt = kt_ref[...]
    qa_ref[...] = _dot3(q, a_ref[...])
    q2 = jnp.sum(q * q, axis=1, keepdims=True)
    k2 = jnp.sum(kt * kt, axis=0, keepdims=True)
    d2 = q2 + k2 - 2.0 * jnp.dot(q, kt, preferred_element_type=jnp.float32)
    big_i = jnp.int32(2 ** 30)
    inf = jnp.float32(jnp.inf)

    m1 = jnp.full((QB, _LANES), inf, jnp.float32)
    m2 = m1
    m3 = m1
    a1 = jnp.zeros((QB, _LANES), jnp.int32)
    a2 = a1
    a3 = a1
    for c in range(_DEPTH):
        x = d2[:, c * _LANES:(c + 1) * _LANES]
        ci = jnp.int32(c)
        u1 = x < m1
        u2 = x < m2
        u3 = x < m3
        m3 = jnp.where(u3, jnp.where(u2, m2, x), m3)
        a3 = jnp.where(u3, jnp.where(u2, a2, ci), a3)
        m2 = jnp.where(u2, jnp.where(u1, m1, x), m2)
        a2 = jnp.where(u2, jnp.where(u1, a1, ci), a2)
        m1 = jnp.where(u1, x, m1)
        a1 = jnp.where(u1, ci, a1)

    v = jnp.concatenate([m1, m2, m3], axis=1)
    a = jnp.concatenate([a1, a2, a3], axis=1)
    lanes = lax.broadcasted_iota(jnp.int32, (QB, 3 * _LANES), 1) % _LANES
    kidx = a * _LANES + lanes
    ixs = []
    d2s = []
    for _ in range(KNN):
        m = jnp.min(v, axis=1, keepdims=True)
        km = jnp.min(jnp.where(v == m, kidx, big_i), axis=1, keepdims=True)
        ixs.append(km)
        d2s.append(m)
        v = jnp.where((v == m) & (kidx == km), inf, v)
    idx_ref[...] = jnp.concatenate(ixs, axis=1)
    d2_ref[...] = jnp.concatenate(d2s, axis=1)

    drained = jnp.any((v[:, 0:_LANES] == inf)
                      & (v[:, _LANES:2 * _LANES] == inf)
                      & (v[:, 2 * _LANES:3 * _LANES] == inf))

    @pl.when(drained)
    def _fallback():
        cols = lax.broadcasted_iota(jnp.int32, (QB, NK), 1)
        dd = d2
        fi = []
        fd = []
        for _ in range(KNN):
            m = jnp.min(dd, axis=1, keepdims=True)
            ix = jnp.min(jnp.where(dd == m, cols, big_i), axis=1,
                         keepdims=True)
            fi.append(ix)
            fd.append(m)
            dd = jnp.where(cols == ix, inf, dd)
        idx_ref[...] = jnp.concatenate(fi, axis=1)
        d2_ref[...] = jnp.concatenate(fd, axis=1)


def _p2(pos_l0, kt, w1a):
    grid = NQ // QB
    return pl.pallas_call(
        _p2_body,
        grid=(grid,),
        in_specs=[
            pl.BlockSpec((QB, 3), lambda i: (i, 0)),
            pl.BlockSpec((3, NK), lambda i: (0, 0)),
            pl.BlockSpec((3, 64), lambda i: (0, 0)),
        ],
        out_specs=(
            pl.BlockSpec((QB, KNN), lambda i: (i, 0)),
            pl.BlockSpec((QB, KNN), lambda i: (i, 0)),
            pl.BlockSpec((QB, 64), lambda i: (i, 0)),
        ),
        out_shape=(
            jax.ShapeDtypeStruct((NQ, KNN), jnp.int32),
            jax.ShapeDtypeStruct((NQ, KNN), jnp.float32),
            jax.ShapeDtypeStruct((NQ, 64), jnp.float32),
        ),
    )(pos_l0, kt, w1a)


def _p3(idx_flat, table):
    info = plsc.get_sparse_core_info()
    nw = info.num_cores * info.num_subcores
    ch = 128
    per_w = NPAIR // (nw * ch)
    mesh = plsc.VectorSubcoreMesh(core_axis_name="c", subcore_axis_name="s")

    @functools.partial(
        pl.kernel,
        out_type=jax.ShapeDtypeStruct((NPAIR, 64), jnp.float32),
        mesh=mesh,
        scratch_types=[
            pltpu.VMEM((ch,), jnp.int32),
            pltpu.VMEM((ch, 64), jnp.float32),
            pltpu.SemaphoreType.DMA,
        ],
        compiler_params=pltpu.CompilerParams(use_tc_tiling_on_sc=False),
    )
    def gather_rows(idx_hbm, tab_hbm, out_hbm, idx_v, rows_v, sem):
        wid = lax.axis_index("s") * info.num_cores + lax.axis_index("c")

        def body(j, carry):
            base = (wid * per_w + j) * ch
            pltpu.sync_copy(idx_hbm.at[pl.ds(base, ch)], idx_v)
            pltpu.async_copy(tab_hbm.at[idx_v], rows_v, sem).wait()
            pltpu.sync_copy(rows_v, out_hbm.at[pl.ds(base, ch)])
            return carry

        lax.fori_loop(0, per_w, body, 0)

    return gather_rows(idx_flat, table)


def _p4_body(gg_ref, qa_ref, d2_ref, w2_ref, b2_ref, w3_ref, b3_ref,
             w4_ref, b4_ref, out_ref):
    qa = qa_ref[...]
    pooled = None
    for n in range(KNN):
        gn = gg_ref[n]
        h1 = jnp.maximum(gn - qa, 0.0)
        h2 = jnp.maximum(jnp.dot(h1, w2_ref[...],
                                 preferred_element_type=jnp.float32, precision=jax.lax.Precision.HIGHEST)
                         + b2_ref[...], 0.0)
        h3 = jnp.maximum(jnp.dot(h2, w3_ref[...],
                                 preferred_element_type=jnp.float32, precision=jax.lax.Precision.HIGHEST)
                         + b3_ref[...], 0.0)
        hm = jnp.where(d2_ref[n] <= 100.0 * 100.0, h3, -1e9)
        pooled = hm if pooled is None else jnp.maximum(pooled, hm)
    out_ref[...] = jnp.maximum(jnp.dot(pooled, w4_ref[...],
                                       preferred_element_type=jnp.float32, precision=jax.lax.Precision.HIGHEST)
                               + b4_ref[...], 0.0)


def _p4(gathered, qa, d2t, w2, b2, w3, b3, w4, b4):
    grid = NQ // PB
    return pl.pallas_call(
        _p4_body,
        grid=(grid,),
        in_specs=[
            pl.BlockSpec((KNN, PB, 64), lambda i: (0, i, 0)),
            pl.BlockSpec((PB, 64), lambda i: (i, 0)),
            pl.BlockSpec((KNN, PB, 1), lambda i: (0, i, 0)),
            pl.BlockSpec((64, 64), lambda i: (0, 0)),
            pl.BlockSpec((1, 64), lambda i: (0, 0)),
            pl.BlockSpec((64, 32), lambda i: (0, 0)),
            pl.BlockSpec((1, 32), lambda i: (0, 0)),
            pl.BlockSpec((32, 32), lambda i: (0, 0)),
            pl.BlockSpec((1, 32), lambda i: (0, 0)),
        ],
        out_specs=pl.BlockSpec((PB, 32), lambda i: (i, 0)),
        out_shape=jax.ShapeDtypeStruct((NQ, 32), jnp.float32),
    )(gathered, qa, d2t, w2, b2.reshape(1, 64), w3, b3.reshape(1, 32),
      w4, b4.reshape(1, 32))


def kernel(pos1_bhw3, pos2_bhw3, color1, color2, intrinsics, occ_mask,
           W_feat, b_feat, W1, b1, W2, b2, W3, b3, W4, b4):
    pos_l0_img = pos1_bhw3[0, ::2, ::4]
    pos_l0 = pos_l0_img.reshape(-1, 3)
    pos_l1 = pos_l0_img[::2, ::2].reshape(-1, 3)
    col_l1 = color1[0, ::2, ::4][::2, ::2].reshape(-1, 3)

    w1a = W1[:3]
    w1b = W1[3:]

    g1t = _p1(col_l1, pos_l1, W_feat, b_feat, w1a, w1b, b1)
    idx, d2t, qa = _p2(pos_l0, pos_l1.T, w1a)
    idx_nm = idx.T.reshape(-1)
    d2_nm = d2t.T.reshape(KNN, NQ, 1)
    gathered = _p3(idx_nm, g1t).reshape(KNN, NQ, 64)
    out = _p4(gathered, qa, d2_nm, W2, b2, W3, b3, W4, b4)
    return out.reshape(1, NQ, 32)
